# baseline (speedup 1.0000x reference)
import jax
import jax.numpy as jnp
from jax import lax
from jax.experimental import pallas as pl
from jax.experimental.pallas import tpu as pltpu
from jax.experimental.pallas import tpu_sc as plsc

_N = 10000
_NP = 10240
_E = 320000
_NC = 2
_NT = 16
_EPW = _E // (_NC * _NT)
_CB = 128
_NCHUNK = 80
_EPAD = _NCHUNK * _CB - _EPW
_RPT = _NP // _NT
_DUMP = 16
_DBASE = _NP - _DUMP
_ZR = 32


def _mesh():
    return plsc.VectorSubcoreMesh(core_axis_name="core", subcore_axis_name="sub")


def _sc_deg():
    scratch = dict(
        ACC=pltpu.VMEM_SHARED((_NP, 16), jnp.float32),
        sidx=pltpu.VMEM((_NCHUNK, _CB), jnp.int32),
        abuf=pltpu.VMEM((_RPT, 16), jnp.float32),
        gbuf=pltpu.VMEM((_CB, 16), jnp.float32),
        zb=pltpu.VMEM((_ZR, 16), jnp.float32),
    )

    def body(srcr, deg_out, ACC, sidx, abuf, gbuf, zb):
        c = lax.axis_index("core")
        t = lax.axis_index("sub")
        r0 = t * _RPT
        zero16 = jnp.zeros((16,), jnp.float32)

        pltpu.sync_copy(srcr.at[c, t], sidx)

        @plsc.parallel_loop(0, _ZR, unroll=8)
        def zb_fill(i):
            zb[i] = zero16

        @plsc.parallel_loop(0, _CB, unroll=8)
        def ones_fill(i):
            gbuf[i] = jnp.full((16,), 1.0, jnp.float32)

        def zbody(m, _):
            pltpu.sync_copy(zb, ACC.at[pl.ds(r0 + m * _ZR, _ZR)])
            return 0
        lax.fori_loop(0, _RPT // _ZR, zbody, 0)
        plsc.subcore_barrier()

        def deg_body(j, _):
            pltpu.sync_copy(gbuf, ACC.at[sidx.at[j]], add=True)
            return 0
        lax.fori_loop(0, _NCHUNK, deg_body, 0)
        plsc.subcore_barrier()

        pltpu.sync_copy(ACC.at[pl.ds(r0, _RPT)], abuf)
        pltpu.sync_copy(abuf, deg_out.at[c, pl.ds(r0, _RPT)])

    return pl.kernel(
        body,
        out_type=[jax.ShapeDtypeStruct((_NC, _NP, 16), jnp.float32)],
        mesh=_mesh(),
        scratch_types=list(scratch.values()),
        compiler_params=pltpu.CompilerParams(use_tc_tiling_on_sc=False),
    )


def _sc_prop(mode):
    scratch = dict(
        G=pltpu.VMEM_SHARED((_NP, 16), jnp.float32),
        ACC=pltpu.VMEM_SHARED((_NP, 16), jnp.float32),
        sidx=pltpu.VMEM((_NCHUNK, _CB), jnp.int32),
        didx=pltpu.VMEM((_NCHUNK, _CB), jnp.int32),
        ybuf=pltpu.VMEM((_RPT, 16), jnp.float32),
        pbuf=pltpu.VMEM((_RPT, 16), jnp.float32),
        dbuf=pltpu.VMEM((_RPT, 16), jnp.float32),
        abuf=pltpu.VMEM((_RPT, 16), jnp.float32),
        gbuf0=pltpu.VMEM((_CB, 16), jnp.float32),
        gbuf1=pltpu.VMEM((_CB, 16), jnp.float32),
        zb=pltpu.VMEM((_ZR, 16), jnp.float32),
        gsem0=pltpu.SemaphoreType.DMA,
        gsem1=pltpu.SemaphoreType.DMA,
        ssem0=pltpu.SemaphoreType.DMA,
        ssem1=pltpu.SemaphoreType.DMA,
    )

    def body(*refs):
        if mode == 1:
            (srcr, dstr, y, d8_in, p_out,
             G, ACC, sidx, didx, ybuf, pbuf, dbuf, abuf, gbuf0, gbuf1, zb,
             gsem0, gsem1, ssem0, ssem1) = refs
        else:
            (srcr, dstr, y, d8_in, p_in, p_out,
             G, ACC, sidx, didx, ybuf, pbuf, dbuf, abuf, gbuf0, gbuf1, zb,
             gsem0, gsem1, ssem0, ssem1) = refs

        c = lax.axis_index("core")
        t = lax.axis_index("sub")
        r0 = t * _RPT
        zero16 = jnp.zeros((16,), jnp.float32)

        pltpu.sync_copy(srcr.at[c, t], sidx)
        pltpu.sync_copy(dstr.at[c, t], didx)
        pltpu.sync_copy(d8_in.at[pl.ds(r0, _RPT)], dbuf)

        @plsc.parallel_loop(0, _ZR, unroll=8)
        def zb_fill(i):
            zb[i] = zero16

        if mode == 1:
            pltpu.sync_copy(y.at[3, pl.ds(r0, _RPT)], ybuf)

            @plsc.parallel_loop(0, _RPT, unroll=8)
            def gb(i):
                abuf[i] = dbuf[i] * ybuf[i]
        elif mode == 2:
            pltpu.sync_copy(y.at[2, pl.ds(r0, _RPT)], ybuf)
            pltpu.sync_copy(p_in.at[0, pl.ds(r0, _RPT)], abuf)
            pltpu.sync_copy(p_in.at[1, pl.ds(r0, _RPT)], pbuf)

            @plsc.parallel_loop(0, _RPT, unroll=8)
            def gb(i):
                d = dbuf[i]
                v = ybuf[i] - 2.0 * d * (abuf[i] + pbuf[i])
                abuf[i] = d * v
        else:
            pltpu.sync_copy(y.at[1, pl.ds(r0, _RPT)], ybuf)
            pltpu.sync_copy(p_in.at[0, pl.ds(r0, _RPT)], abuf)
            pltpu.sync_copy(p_in.at[1, pl.ds(r0, _RPT)], pbuf)

            @plsc.parallel_loop(0, _RPT, unroll=8)
            def gb1(i):
                pbuf[i] = abuf[i] + pbuf[i]

            pltpu.sync_copy(y.at[3, pl.ds(r0, _RPT)], abuf)

            @plsc.parallel_loop(0, _RPT, unroll=8)
            def gb(i):
                d = dbuf[i]
                v = ybuf[i] - 2.0 * d * pbuf[i] - abuf[i]
                abuf[i] = d * v

        pltpu.sync_copy(abuf, G.at[pl.ds(r0, _RPT)])

        def zbody(m, _):
            pltpu.sync_copy(zb, ACC.at[pl.ds(r0 + m * _ZR, _ZR)])
            return 0
        lax.fori_loop(0, _RPT // _ZR, zbody, 0)
        plsc.subcore_barrier()

        def chunk_pair(jj, _):
            j0 = 2 * jj
            j1 = j0 + 1

            @pl.when(jj > 0)
            def _():
                pltpu.make_async_copy(
                    gbuf0, ACC.at[didx.at[j0]], ssem0).wait()
                pltpu.make_async_copy(
                    gbuf1, ACC.at[didx.at[j1]], ssem1).wait()

            g0 = pltpu.async_copy(G.at[sidx.at[j0]], gbuf0, gsem0)
            g1 = pltpu.async_copy(G.at[sidx.at[j1]], gbuf1, gsem1)
            g0.wait()
            pltpu.async_copy(gbuf0, ACC.at[didx.at[j0]], ssem0, add=True)
            g1.wait()
            pltpu.async_copy(gbuf1, ACC.at[didx.at[j1]], ssem1, add=True)
            return 0
        lax.fori_loop(0, _NCHUNK // 2, chunk_pair, 0)
        pltpu.make_async_copy(gbuf0, ACC.at[didx.at[0]], ssem0).wait()
        pltpu.make_async_copy(gbuf1, ACC.at[didx.at[1]], ssem1).wait()
        plsc.subcore_barrier()

        pltpu.sync_copy(ACC.at[pl.ds(r0, _RPT)], abuf)
        pltpu.sync_copy(abuf, p_out.at[c, pl.ds(r0, _RPT)])

    return pl.kernel(
        body,
        out_type=[jax.ShapeDtypeStruct((_NC, _NP, 16), jnp.float32)],
        mesh=_mesh(),
        scratch_types=list(scratch.values()),
        compiler_params=pltpu.CompilerParams(use_tc_tiling_on_sc=False),
    )


def _mm1(x, wcat, bcat, deg, blk=1024):
    n, kd = x.shape

    def bodyf(x_ref, w_ref, b_ref, deg_ref, o_ref, d8_ref):
        acc = jnp.dot(x_ref[...], w_ref[...],
                      preferred_element_type=jnp.float32) + b_ref[...]
        for k in range(4):
            o_ref[k] = acc[:, k * 16:(k + 1) * 16]
        dv = deg_ref[0] + deg_ref[1]
        d8_ref[...] = jnp.where(dv > 0.5, lax.rsqrt(dv), 0.0)

    return pl.pallas_call(
        bodyf,
        grid=(n // blk,),
        in_specs=[
            pl.BlockSpec((blk, kd), lambda i: (i, 0)),
            pl.BlockSpec((kd, 64), lambda i: (0, 0)),
            pl.BlockSpec((1, 64), lambda i: (0, 0)),
            pl.BlockSpec((_NC, blk, 16), lambda i: (0, i, 0)),
        ],
        out_specs=[
            pl.BlockSpec((4, blk, 16), lambda i: (0, i, 0)),
            pl.BlockSpec((blk, 16), lambda i: (i, 0)),
        ],
        out_shape=[
            jax.ShapeDtypeStruct((4, n, 16), jnp.float32),
            jax.ShapeDtypeStruct((n, 16), jnp.float32),
        ],
    )(x, wcat, bcat, deg)


def _finish1_mm2(y1, p1, p3, d8, wcat, bcat, blk=1024):
    n = y1.shape[1]

    def bodyf(y_ref, p1_ref, p3_ref, d8_ref, w_ref, b_ref, o_ref):
        dv = d8_ref[...]
        b2s = y_ref[2] - 2.0 * dv * (p1_ref[0] + p1_ref[1])
        h = y_ref[0] - dv * (p3_ref[0] + p3_ref[1]) - b2s
        h = jnp.maximum(h, 0.0)
        acc = jnp.dot(h, w_ref[...],
                      preferred_element_type=jnp.float32) + b_ref[...]
        for k in range(4):
            o_ref[k] = acc[:, k * 16:(k + 1) * 16]

    return pl.pallas_call(
        bodyf,
        grid=(n // blk,),
        in_specs=[
            pl.BlockSpec((4, blk, 16), lambda i: (0, i, 0)),
            pl.BlockSpec((_NC, blk, 16), lambda i: (0, i, 0)),
            pl.BlockSpec((_NC, blk, 16), lambda i: (0, i, 0)),
            pl.BlockSpec((blk, 16), lambda i: (i, 0)),
            pl.BlockSpec((16, 64), lambda i: (0, 0)),
            pl.BlockSpec((1, 64), lambda i: (0, 0)),
        ],
        out_specs=pl.BlockSpec((4, blk, 16), lambda i: (0, i, 0)),
        out_shape=jax.ShapeDtypeStruct((4, n, 16), jnp.float32),
    )(y1, p1, p3, d8, wcat, bcat)


def _finish2_softmax(y2, p1, p3, d8, blk=1024):
    n = y2.shape[1]

    def bodyf(y_ref, p1_ref, p3_ref, d8_ref, o_ref):
        dv = d8_ref[...]
        b2s = y_ref[2] - 2.0 * dv * (p1_ref[0] + p1_ref[1])
        xv = y_ref[0] - dv * (p3_ref[0] + p3_ref[1]) - b2s
        col = lax.broadcasted_iota(jnp.int32, xv.shape, 1)
        xm = jnp.where(col < 10, xv, -1e30)
        m = jnp.max(xm, axis=1, keepdims=True)
        e = jnp.exp(xm - m)
        lse = jnp.log(jnp.sum(e, axis=1, keepdims=True)) + m
        o_ref[...] = xv[:, :10] - lse

    return pl.pallas_call(
        bodyf,
        grid=(n // blk,),
        in_specs=[
            pl.BlockSpec((4, blk, 16), lambda i: (0, i, 0)),
            pl.BlockSpec((_NC, blk, 16), lambda i: (0, i, 0)),
            pl.BlockSpec((_NC, blk, 16), lambda i: (0, i, 0)),
            pl.BlockSpec((blk, 16), lambda i: (i, 0)),
        ],
        out_specs=pl.BlockSpec((blk, 10), lambda i: (i, 0)),
        out_shape=jax.ShapeDtypeStruct((n, 10), jnp.float32),
    )(y2, p1, p3, d8)


_sc_deg_k = _sc_deg()
_sc_prop1 = _sc_prop(1)
_sc_prop2 = _sc_prop(2)
_sc_prop3 = _sc_prop(3)


def kernel(x, edge_index, W1, b1, W2, b2):
    f_in = x.shape[1]

    pad = (_DBASE + (jnp.arange(_EPAD, dtype=jnp.int32) % _DUMP))
    pad = jnp.broadcast_to(pad[None, :], (_NC * _NT, _EPAD))
    src_r = jnp.concatenate(
        [edge_index[0].reshape(_NC * _NT, _EPW), pad], axis=1
    ).reshape(_NC, _NT, _NCHUNK, _CB)
    dst_r = jnp.concatenate(
        [edge_index[1].reshape(_NC * _NT, _EPW), pad], axis=1
    ).reshape(_NC, _NT, _NCHUNK, _CB)

    xp = jnp.pad(x, ((0, _NP - _N), (0, 0)))

    (deg,) = _sc_deg_k(src_r)

    w1c = W1.transpose(1, 0, 2).reshape(f_in, 64)
    b1c = jnp.concatenate([b1, jnp.zeros((48,), jnp.float32)])[None, :]
    y1, d8 = _mm1(xp, w1c, b1c, deg)
    (p1,) = _sc_prop1(src_r, dst_r, y1, d8)
    (p2,) = _sc_prop2(src_r, dst_r, y1, d8, p1)
    (p3,) = _sc_prop3(src_r, dst_r, y1, d8, p2)

    w2p = jnp.pad(W2, ((0, 0), (0, 0), (0, 6)))
    w2c = w2p.transpose(1, 0, 2).reshape(16, 64)
    b2c = jnp.concatenate([b2, jnp.zeros((54,), jnp.float32)])[None, :]
    y2 = _finish1_mm2(y1, p1, p3, d8, w2c, b2c)
    (q1,) = _sc_prop1(src_r, dst_r, y2, d8)
    (q2,) = _sc_prop2(src_r, dst_r, y2, d8, q1)
    (q3,) = _sc_prop3(src_r, dst_r, y2, d8, q2)

    return _finish2_softmax(y2, q1, q3, d8)[:_N]


# speedup vs baseline: 1.0631x; 1.0631x over previous
import jax
import jax.numpy as jnp
from jax import lax
from jax.experimental import pallas as pl
from jax.experimental.pallas import tpu as pltpu
from jax.experimental.pallas import tpu_sc as plsc

_N = 10000
_NP = 10240
_E = 320000
_NC = 2
_NT = 16
_EPW = _E // (_NC * _NT)
_CB = 128
_NCHUNK = 80
_EPAD = _NCHUNK * _CB - _EPW
_RPT = _NP // _NT
_DUMP = 16
_DBASE = _NP - _DUMP
_ZR = 32


def _mesh():
    return plsc.VectorSubcoreMesh(core_axis_name="core", subcore_axis_name="sub")


def _sc_deg():
    scratch = dict(
        ACC=pltpu.VMEM_SHARED((_NP, 16), jnp.float32),
        sidx=pltpu.VMEM((_NCHUNK, _CB), jnp.int32),
        abuf=pltpu.VMEM((_RPT, 16), jnp.float32),
        gbuf=pltpu.VMEM((_CB, 16), jnp.float32),
        zb=pltpu.VMEM((_ZR, 16), jnp.float32),
    )

    def body(srcr, deg_out, ACC, sidx, abuf, gbuf, zb):
        c = lax.axis_index("core")
        t = lax.axis_index("sub")
        r0 = t * _RPT
        zero16 = jnp.zeros((16,), jnp.float32)

        pltpu.sync_copy(srcr.at[c, t], sidx)

        @plsc.parallel_loop(0, _ZR, unroll=8)
        def zb_fill(i):
            zb[i] = zero16

        @plsc.parallel_loop(0, _CB, unroll=8)
        def ones_fill(i):
            gbuf[i] = jnp.full((16,), 1.0, jnp.float32)

        def zbody(m, _):
            pltpu.sync_copy(zb, ACC.at[pl.ds(r0 + m * _ZR, _ZR)])
            return 0
        lax.fori_loop(0, _RPT // _ZR, zbody, 0)
        plsc.subcore_barrier()

        def deg_body(j, _):
            pltpu.sync_copy(gbuf, ACC.at[sidx.at[j]], add=True)
            return 0
        lax.fori_loop(0, _NCHUNK, deg_body, 0)
        plsc.subcore_barrier()

        pltpu.sync_copy(ACC.at[pl.ds(r0, _RPT)], abuf)
        pltpu.sync_copy(abuf, deg_out.at[c, pl.ds(r0, _RPT)])

    return pl.kernel(
        body,
        out_type=[jax.ShapeDtypeStruct((_NC, _NP, 16), jnp.float32)],
        mesh=_mesh(),
        scratch_types=list(scratch.values()),
        compiler_params=pltpu.CompilerParams(use_tc_tiling_on_sc=False),
    )


def _sc_prop(mode):
    scratch = dict(
        G=pltpu.VMEM_SHARED((_NP, 16), jnp.float32),
        ACC=pltpu.VMEM_SHARED((_NP, 16), jnp.float32),
        sidx=pltpu.VMEM((_NCHUNK, _CB), jnp.int32),
        didx=pltpu.VMEM((_NCHUNK, _CB), jnp.int32),
        ybuf=pltpu.VMEM((_RPT, 16), jnp.float32),
        pbuf=pltpu.VMEM((_RPT, 16), jnp.float32),
        dbuf=pltpu.VMEM((_RPT, 16), jnp.float32),
        abuf=pltpu.VMEM((_RPT, 16), jnp.float32),
        gbuf0=pltpu.VMEM((_CB, 16), jnp.float32),
        gbuf1=pltpu.VMEM((_CB, 16), jnp.float32),
        gbuf2=pltpu.VMEM((_CB, 16), jnp.float32),
        gbuf3=pltpu.VMEM((_CB, 16), jnp.float32),
        zb=pltpu.VMEM((_ZR, 16), jnp.float32),
        gsem0=pltpu.SemaphoreType.DMA,
        gsem1=pltpu.SemaphoreType.DMA,
        gsem2=pltpu.SemaphoreType.DMA,
        gsem3=pltpu.SemaphoreType.DMA,
        ssem0=pltpu.SemaphoreType.DMA,
        ssem1=pltpu.SemaphoreType.DMA,
        ssem2=pltpu.SemaphoreType.DMA,
        ssem3=pltpu.SemaphoreType.DMA,
    )

    def body(*refs):
        if mode == 1:
            (srcr, dstr, y, d8_in, p_out,
             G, ACC, sidx, didx, ybuf, pbuf, dbuf, abuf,
             gbuf0, gbuf1, gbuf2, gbuf3, zb,
             gsem0, gsem1, gsem2, gsem3, ssem0, ssem1, ssem2, ssem3) = refs
        else:
            (srcr, dstr, y, d8_in, p_in, p_out,
             G, ACC, sidx, didx, ybuf, pbuf, dbuf, abuf,
             gbuf0, gbuf1, gbuf2, gbuf3, zb,
             gsem0, gsem1, gsem2, gsem3, ssem0, ssem1, ssem2, ssem3) = refs

        c = lax.axis_index("core")
        t = lax.axis_index("sub")
        r0 = t * _RPT
        zero16 = jnp.zeros((16,), jnp.float32)

        ld_s = pltpu.async_copy(srcr.at[c, t], sidx, gsem0)
        ld_d = pltpu.async_copy(dstr.at[c, t], didx, gsem1)
        ld_8 = pltpu.async_copy(d8_in.at[pl.ds(r0, _RPT)], dbuf, gsem2)

        @plsc.parallel_loop(0, _ZR, unroll=8)
        def zb_fill(i):
            zb[i] = zero16
        ld_8.wait()

        if mode == 1:
            pltpu.sync_copy(y.at[3, pl.ds(r0, _RPT)], ybuf)

            @plsc.parallel_loop(0, _RPT, unroll=8)
            def gb(i):
                abuf[i] = dbuf[i] * ybuf[i]
        elif mode == 2:
            pltpu.sync_copy(y.at[2, pl.ds(r0, _RPT)], ybuf)
            pltpu.sync_copy(p_in.at[0, pl.ds(r0, _RPT)], abuf)
            pltpu.sync_copy(p_in.at[1, pl.ds(r0, _RPT)], pbuf)

            @plsc.parallel_loop(0, _RPT, unroll=8)
            def gb(i):
                d = dbuf[i]
                v = ybuf[i] - 2.0 * d * (abuf[i] + pbuf[i])
                abuf[i] = d * v
        else:
            pltpu.sync_copy(y.at[1, pl.ds(r0, _RPT)], ybuf)
            pltpu.sync_copy(p_in.at[0, pl.ds(r0, _RPT)], abuf)
            pltpu.sync_copy(p_in.at[1, pl.ds(r0, _RPT)], pbuf)

            @plsc.parallel_loop(0, _RPT, unroll=8)
            def gb1(i):
                pbuf[i] = abuf[i] + pbuf[i]

            pltpu.sync_copy(y.at[3, pl.ds(r0, _RPT)], abuf)

            @plsc.parallel_loop(0, _RPT, unroll=8)
            def gb(i):
                d = dbuf[i]
                v = ybuf[i] - 2.0 * d * pbuf[i] - abuf[i]
                abuf[i] = d * v

        pltpu.sync_copy(abuf, G.at[pl.ds(r0, _RPT)])

        def zbody(m, _):
            pltpu.sync_copy(zb, ACC.at[pl.ds(r0 + m * _ZR, _ZR)])
            return 0
        lax.fori_loop(0, _RPT // _ZR, zbody, 0)
        ld_s.wait()
        ld_d.wait()
        plsc.subcore_barrier()

        bufs = (gbuf0, gbuf1, gbuf2, gbuf3)
        gsems = (gsem0, gsem1, gsem2, gsem3)
        ssems = (ssem0, ssem1, ssem2, ssem3)

        def chunk_quad(jj, _):
            j0 = 4 * jj

            @pl.when(jj > 0)
            def _():
                for b in range(4):
                    pltpu.make_async_copy(
                        bufs[b], ACC.at[didx.at[j0 + b]], ssems[b]).wait()

            gds = [
                pltpu.async_copy(G.at[sidx.at[j0 + b]], bufs[b], gsems[b])
                for b in range(4)
            ]
            for b in range(4):
                gds[b].wait()
                pltpu.async_copy(
                    bufs[b], ACC.at[didx.at[j0 + b]], ssems[b], add=True)
            return 0
        lax.fori_loop(0, _NCHUNK // 4, chunk_quad, 0)
        for b in range(4):
            pltpu.make_async_copy(bufs[b], ACC.at[didx.at[b]], ssems[b]).wait()
        plsc.subcore_barrier()

        pltpu.sync_copy(ACC.at[pl.ds(r0, _RPT)], abuf)
        pltpu.sync_copy(abuf, p_out.at[c, pl.ds(r0, _RPT)])

    return pl.kernel(
        body,
        out_type=[jax.ShapeDtypeStruct((_NC, _NP, 16), jnp.float32)],
        mesh=_mesh(),
        scratch_types=list(scratch.values()),
        compiler_params=pltpu.CompilerParams(use_tc_tiling_on_sc=False),
    )


def _mm1(x, wcat, bcat, deg, blk=1024):
    n, kd = x.shape

    def bodyf(x_ref, w_ref, b_ref, deg_ref, o_ref, d8_ref):
        acc = jnp.dot(x_ref[...], w_ref[...],
                      preferred_element_type=jnp.float32) + b_ref[...]
        for k in range(4):
            o_ref[k] = acc[:, k * 16:(k + 1) * 16]
        dv = deg_ref[0] + deg_ref[1]
        d8_ref[...] = jnp.where(dv > 0.5, lax.rsqrt(dv), 0.0)

    return pl.pallas_call(
        bodyf,
        grid=(n // blk,),
        in_specs=[
            pl.BlockSpec((blk, kd), lambda i: (i, 0)),
            pl.BlockSpec((kd, 64), lambda i: (0, 0)),
            pl.BlockSpec((1, 64), lambda i: (0, 0)),
            pl.BlockSpec((_NC, blk, 16), lambda i: (0, i, 0)),
        ],
        out_specs=[
            pl.BlockSpec((4, blk, 16), lambda i: (0, i, 0)),
            pl.BlockSpec((blk, 16), lambda i: (i, 0)),
        ],
        out_shape=[
            jax.ShapeDtypeStruct((4, n, 16), jnp.float32),
            jax.ShapeDtypeStruct((n, 16), jnp.float32),
        ],
    )(x, wcat, bcat, deg)


def _finish1_mm2(y1, p1, p3, d8, wcat, bcat, blk=1024):
    n = y1.shape[1]

    def bodyf(y_ref, p1_ref, p3_ref, d8_ref, w_ref, b_ref, o_ref):
        dv = d8_ref[...]
        b2s = y_ref[2] - 2.0 * dv * (p1_ref[0] + p1_ref[1])
        h = y_ref[0] - dv * (p3_ref[0] + p3_ref[1]) - b2s
        h = jnp.maximum(h, 0.0)
        acc = jnp.dot(h, w_ref[...],
                      preferred_element_type=jnp.float32) + b_ref[...]
        for k in range(4):
            o_ref[k] = acc[:, k * 16:(k + 1) * 16]

    return pl.pallas_call(
        bodyf,
        grid=(n // blk,),
        in_specs=[
            pl.BlockSpec((4, blk, 16), lambda i: (0, i, 0)),
            pl.BlockSpec((_NC, blk, 16), lambda i: (0, i, 0)),
            pl.BlockSpec((_NC, blk, 16), lambda i: (0, i, 0)),
            pl.BlockSpec((blk, 16), lambda i: (i, 0)),
            pl.BlockSpec((16, 64), lambda i: (0, 0)),
            pl.BlockSpec((1, 64), lambda i: (0, 0)),
        ],
        out_specs=pl.BlockSpec((4, blk, 16), lambda i: (0, i, 0)),
        out_shape=jax.ShapeDtypeStruct((4, n, 16), jnp.float32),
    )(y1, p1, p3, d8, wcat, bcat)


def _finish2_softmax(y2, p1, p3, d8, blk=1024):
    n = y2.shape[1]

    def bodyf(y_ref, p1_ref, p3_ref, d8_ref, o_ref):
        dv = d8_ref[...]
        b2s = y_ref[2] - 2.0 * dv * (p1_ref[0] + p1_ref[1])
        xv = y_ref[0] - dv * (p3_ref[0] + p3_ref[1]) - b2s
        col = lax.broadcasted_iota(jnp.int32, xv.shape, 1)
        xm = jnp.where(col < 10, xv, -1e30)
        m = jnp.max(xm, axis=1, keepdims=True)
        e = jnp.exp(xm - m)
        lse = jnp.log(jnp.sum(e, axis=1, keepdims=True)) + m
        o_ref[...] = xv[:, :10] - lse

    return pl.pallas_call(
        bodyf,
        grid=(n // blk,),
        in_specs=[
            pl.BlockSpec((4, blk, 16), lambda i: (0, i, 0)),
            pl.BlockSpec((_NC, blk, 16), lambda i: (0, i, 0)),
            pl.BlockSpec((_NC, blk, 16), lambda i: (0, i, 0)),
            pl.BlockSpec((blk, 16), lambda i: (i, 0)),
        ],
        out_specs=pl.BlockSpec((blk, 10), lambda i: (i, 0)),
        out_shape=jax.ShapeDtypeStruct((n, 10), jnp.float32),
    )(y2, p1, p3, d8)


_sc_deg_k = _sc_deg()
_sc_prop1 = _sc_prop(1)
_sc_prop2 = _sc_prop(2)
_sc_prop3 = _sc_prop(3)


def kernel(x, edge_index, W1, b1, W2, b2):
    f_in = x.shape[1]

    pad = (_DBASE + (jnp.arange(_EPAD, dtype=jnp.int32) % _DUMP))
    pad = jnp.broadcast_to(pad[None, :], (_NC * _NT, _EPAD))
    src_r = jnp.concatenate(
        [edge_index[0].reshape(_NC * _NT, _EPW), pad], axis=1
    ).reshape(_NC, _NT, _NCHUNK, _CB)
    dst_r = jnp.concatenate(
        [edge_index[1].reshape(_NC * _NT, _EPW), pad], axis=1
    ).reshape(_NC, _NT, _NCHUNK, _CB)

    xp = jnp.pad(x, ((0, _NP - _N), (0, 0)))

    (deg,) = _sc_deg_k(src_r)

    w1c = W1.transpose(1, 0, 2).reshape(f_in, 64)
    b1c = jnp.concatenate([b1, jnp.zeros((48,), jnp.float32)])[None, :]
    y1, d8 = _mm1(xp, w1c, b1c, deg)
    (p1,) = _sc_prop1(src_r, dst_r, y1, d8)
    (p2,) = _sc_prop2(src_r, dst_r, y1, d8, p1)
    (p3,) = _sc_prop3(src_r, dst_r, y1, d8, p2)

    w2p = jnp.pad(W2, ((0, 0), (0, 0), (0, 6)))
    w2c = w2p.transpose(1, 0, 2).reshape(16, 64)
    b2c = jnp.concatenate([b2, jnp.zeros((54,), jnp.float32)])[None, :]
    y2 = _finish1_mm2(y1, p1, p3, d8, w2c, b2c)
    (q1,) = _sc_prop1(src_r, dst_r, y2, d8)
    (q2,) = _sc_prop2(src_r, dst_r, y2, d8, q1)
    (q3,) = _sc_prop3(src_r, dst_r, y2, d8, q2)

    return _finish2_softmax(y2, q1, q3, d8)[:_N]
